# baseline (speedup 1.0000x reference)
import functools

import jax
import jax.numpy as jnp
from jax import lax
from jax.experimental import pallas as pl
from jax.experimental.pallas import tpu as pltpu
from jax.experimental.pallas import tpu_sc as plsc

_N = 10000
_E = 320000
_LD = 128
_OUT = 3
_R = 16
_BITWAVES = (4, 4)
_BQ = 4

_INTERPRET = False


def _dot(a, b):
    return jnp.dot(a, b, preferred_element_type=jnp.float32)


def _encode_body(x_ref, Wn_ref, bn_ref, h_ref):
    h_ref[...] = _dot(x_ref[...], Wn_ref[...]) + bn_ref[...]


_BN = 2000
_NBN = _N // _BN


def _encode(x, Wn, bn):
    return pl.pallas_call(
        _encode_body,
        grid=(_NBN,),
        in_specs=[
            pl.BlockSpec((_BN, _LD), lambda i: (i, 0)),
            pl.BlockSpec((_LD, _LD), lambda i: (0, 0)),
            pl.BlockSpec((1, _LD), lambda i: (0, 0)),
        ],
        out_specs=pl.BlockSpec((_BN, _LD), lambda i: (i, 0)),
        out_shape=jax.ShapeDtypeStruct((_N, _LD), jnp.float32),
        interpret=_INTERPRET,
    )(x, Wn, bn.reshape(1, _LD))


def _pre_body(h_ref, acc_ref, Wa_ref, ba_ref, A_ref, B_ref, C_ref, D_ref,
              ST_ref, acch_ref):
    pid = pl.program_id(0)
    h = h_ref[...]
    acc_h = _dot(acc_ref[...], Wa_ref[...]) + ba_ref[...]
    acch_ref[...] = acc_h

    @pl.when(pid < _NBN)
    def _():
        ST_ref[...] = _dot(h, A_ref[...]) + _dot(acc_h, C_ref[...])

    @pl.when(pid >= _NBN)
    def _():
        ST_ref[...] = _dot(h, B_ref[...]) + _dot(acc_h, D_ref[...])


def _pre(h, acc, Wa, ba, A, B, C, D):
    return pl.pallas_call(
        _pre_body,
        grid=(2 * _NBN,),
        in_specs=[
            pl.BlockSpec((_BN, _LD), lambda i: (i % _NBN, 0)),
            pl.BlockSpec((_BN, _OUT), lambda i: (i % _NBN, 0)),
            pl.BlockSpec((_OUT, _LD), lambda i: (0, 0)),
            pl.BlockSpec((1, _LD), lambda i: (0, 0)),
            pl.BlockSpec((_LD, _LD), lambda i: (0, 0)),
            pl.BlockSpec((_LD, _LD), lambda i: (0, 0)),
            pl.BlockSpec((_LD, _LD), lambda i: (0, 0)),
            pl.BlockSpec((_LD, _LD), lambda i: (0, 0)),
        ],
        out_specs=(pl.BlockSpec((_BN, _LD), lambda i: (i, 0)),
                   pl.BlockSpec((_BN, _LD), lambda i: (i % _NBN, 0))),
        out_shape=(jax.ShapeDtypeStruct((2 * _N, _LD), jnp.float32),
                   jax.ShapeDtypeStruct((_N, _LD), jnp.float32)),
        interpret=_INTERPRET,
    )(h, acc, Wa, ba.reshape(1, _LD), A, B, C, D)


def _edge_encode_body(ea_ref, We_ref, be_ref, eh_ref):
    eh_ref[...] = _dot(ea_ref[...], We_ref[...]) + be_ref[...]


def _edge_encode(edge_attr, We, be):
    BE = 2560
    return pl.pallas_call(
        _edge_encode_body,
        grid=(_E // BE,),
        in_specs=[
            pl.BlockSpec((BE, 16), lambda i: (i, 0)),
            pl.BlockSpec((16, _LD), lambda i: (0, 0)),
            pl.BlockSpec((1, _LD), lambda i: (0, 0)),
        ],
        out_specs=pl.BlockSpec((BE, _LD), lambda i: (i, 0)),
        out_shape=jax.ShapeDtypeStruct((_E, _LD), jnp.float32),
        interpret=_INTERPRET,
    )(edge_attr, We, be.reshape(1, _LD))


def _edge_body(GS_ref, GT_ref, eh_ref, F_ref, bm1_ref, Wm2_ref, bm2_ref,
               m_ref):
    hidden = jnp.maximum(
        GS_ref[...] + GT_ref[...] + _dot(eh_ref[...], F_ref[...])
        + bm1_ref[...], 0.0)
    m_ref[...] = jnp.maximum(_dot(hidden, Wm2_ref[...]) + bm2_ref[...], 0.0)


def _edge_mlp(Gst, eh, F, bm1, Wm2, bm2):
    BE = 2560
    nblk = _E // BE
    grid = (nblk,)
    return pl.pallas_call(
        _edge_body,
        grid=grid,
        in_specs=[
            pl.BlockSpec((BE, _LD), lambda i: (i, 0)),
            pl.BlockSpec((BE, _LD), lambda i: (i + nblk, 0)),
            pl.BlockSpec((BE, _LD), lambda i: (i, 0)),
            pl.BlockSpec((_LD, _LD), lambda i: (0, 0)),
            pl.BlockSpec((1, _LD), lambda i: (0, 0)),
            pl.BlockSpec((_LD, _LD), lambda i: (0, 0)),
            pl.BlockSpec((1, _LD), lambda i: (0, 0)),
        ],
        out_specs=pl.BlockSpec((BE, _LD), lambda i: (i, 0)),
        out_shape=jax.ShapeDtypeStruct((_E, _LD), jnp.float32),
        interpret=_INTERPRET,
    )(Gst, Gst, eh, F, bm1.reshape(1, _LD), Wm2, bm2.reshape(1, _LD))


def _post_body(h_ref, acch_ref, acc_ref, agg0_ref, agg1_ref, Wu1a_ref,
               Wu1b_ref, Wu1c_ref, bu1_ref, Wu2_ref, bu2_ref, g_ref, b_ref,
               Wd_ref, bd_ref, h1_ref, acc1_ref, logits_ref, *, scale):
    agg = agg0_ref[...] + agg1_ref[...]
    u = jnp.maximum(
        _dot(h_ref[...], Wu1a_ref[...]) + _dot(acch_ref[...], Wu1b_ref[...])
        + _dot(agg, Wu1c_ref[...]) + bu1_ref[...], 0.0)
    hn = _dot(u, Wu2_ref[...]) + bu2_ref[...]
    mu = jnp.mean(hn, axis=1, keepdims=True)
    hc = hn - mu
    var = jnp.mean(hc * hc, axis=1, keepdims=True)
    hn = hc / jnp.sqrt(var + 1e-5) * g_ref[...] + b_ref[...]
    h1_ref[...] = hn
    logits = _dot(hn, Wd_ref[...]) + bd_ref[...]
    logits_ref[...] = logits
    cols = []
    for j in range(_OUT):
        lj = logits[:, j * _R:(j + 1) * _R]
        mx = jnp.max(lj, axis=1, keepdims=True)
        iota = lax.broadcasted_iota(jnp.int32, lj.shape, 1)
        idx = jnp.min(jnp.where(lj == mx, iota, _R), axis=1, keepdims=True)
        cols.append(idx)
    bits = jnp.concatenate(cols, axis=1).astype(jnp.float32)
    acc1_ref[...] = acc_ref[...] + bits * scale


def _post(h, acc_h, acc, agg_p, Wu1a, Wu1b, Wu1c, bu1, Wu2, bu2, g, b, Wd, bd,
          scale):
    outs = (
        jax.ShapeDtypeStruct((_N, _LD), jnp.float32),
        jax.ShapeDtypeStruct((_N, _OUT), jnp.float32),
        jax.ShapeDtypeStruct((_N, _OUT * _R), jnp.float32),
    )
    full = lambda r, c: pl.BlockSpec((r, c), lambda i: (0, 0))
    return pl.pallas_call(
        functools.partial(_post_body, scale=scale),
        grid=(_NBN,),
        in_specs=[
            pl.BlockSpec((_BN, _LD), lambda i: (i, 0)),
            pl.BlockSpec((_BN, _LD), lambda i: (i, 0)),
            pl.BlockSpec((_BN, _OUT), lambda i: (i, 0)),
            pl.BlockSpec((_BN, _LD), lambda i: (i, 0)),
            pl.BlockSpec((_BN, _LD), lambda i: (i + _NBN, 0)),
            full(_LD, _LD), full(_LD, _LD), full(_LD, _LD), full(1, _LD),
            full(_LD, _LD), full(1, _LD), full(1, _LD), full(1, _LD),
            full(_LD, _OUT * _R), full(1, _OUT * _R),
        ],
        out_specs=(pl.BlockSpec((_BN, _LD), lambda i: (i, 0)),
                   pl.BlockSpec((_BN, _OUT), lambda i: (i, 0)),
                   pl.BlockSpec((_BN, _OUT * _R), lambda i: (i, 0))),
        out_shape=outs,
        interpret=_INTERPRET,
    )(h, acc_h, acc, agg_p, agg_p, Wu1a, Wu1b, Wu1c, bu1.reshape(1, _LD),
      Wu2, bu2.reshape(1, _LD), g.reshape(1, _LD), b.reshape(1, _LD), Wd,
      bd.reshape(1, _OUT * _R))


_NC = 2
_NS = 16
_NW = _NC * _NS
_GC = 128


def _sc_mesh():
    return plsc.VectorSubcoreMesh(core_axis_name="c", subcore_axis_name="s")


_GCH = (2 * _E) // _GC
_GFULL = _GCH // _NW
_GEXTRA = _GCH - _GFULL * _NW


_GNB = 4


def _sc_gather_body(table_hbm, idx_hbm, out_hbm, idx_v, rows0, rows1, rows2,
                    rows3, sem0, sem1, sem2, sem3):
    rows = (rows0, rows1, rows2, rows3)
    sems = (sem0, sem1, sem2, sem3)
    wid = lax.axis_index("s") * _NC + lax.axis_index("c")
    cstart = wid * _GFULL + jnp.minimum(wid, _GEXTRA)
    pltpu.sync_copy(idx_hbm.at[pl.ds(cstart * _GC, _GFULL * _GC)],
                    idx_v.at[pl.ds(0, _GFULL * _GC)])

    @pl.when(wid < _GEXTRA)
    def _():
        pltpu.sync_copy(idx_hbm.at[pl.ds((cstart + _GFULL) * _GC, _GC)],
                        idx_v.at[pl.ds(_GFULL * _GC, _GC)])

    def _idx(k):
        return idx_v.at[pl.ds(k * _GC, _GC)]

    def _out(k):
        return out_hbm.at[pl.ds((cstart + k) * _GC, _GC)]

    for b in range(_GNB - 1):
        pltpu.async_copy(table_hbm.at[_idx(b)], rows[b], sems[b])

    @pl.loop(0, _GFULL, step=_GNB)
    def _(k):
        for b in range(_GNB):
            jb = k + b
            pltpu.make_async_copy(table_hbm.at[_idx(jb)], rows[b],
                                  sems[b]).wait()
            nb = (b + _GNB - 1) % _GNB

            @pl.when(jb + _GNB - 1 < _GFULL)
            def _(jb=jb, nb=nb):
                pltpu.async_copy(table_hbm.at[_idx(jb + _GNB - 1)], rows[nb],
                                 sems[nb])

            pltpu.sync_copy(rows[b], _out(jb))

    @pl.when(wid < _GEXTRA)
    def _():
        pltpu.async_copy(table_hbm.at[_idx(_GFULL)], rows0, sem0).wait()
        pltpu.sync_copy(rows0, _out(_GFULL))


def _sc_gather(ST, idx2):
    return pl.kernel(
        _sc_gather_body,
        out_type=jax.ShapeDtypeStruct((2 * _E, _LD), jnp.float32),
        mesh=_sc_mesh(),
        scratch_types=[
            pltpu.VMEM(((_GFULL + 1) * _GC,), jnp.int32),
            pltpu.VMEM((_GC, _LD), jnp.float32),
            pltpu.VMEM((_GC, _LD), jnp.float32),
            pltpu.VMEM((_GC, _LD), jnp.float32),
            pltpu.VMEM((_GC, _LD), jnp.float32),
            pltpu.SemaphoreType.DMA,
            pltpu.SemaphoreType.DMA,
            pltpu.SemaphoreType.DMA,
            pltpu.SemaphoreType.DMA,
        ],
    )(ST, idx2)


_SCH = _E // _GC
_SPW = 80
_SLAST = _SCH - _SPW * (_NW - 1)


def _sc_scatter_body(m_hbm, dst2d_hbm, zeros_hbm, out_hbm, idx_v, rows0,
                     rows1, acc_sh, sem0, sem1):
    cid = lax.axis_index("c")
    sid = lax.axis_index("s")
    slab = 632
    last = _N - (_NS - 1) * slab
    zoff = sid * slab

    @pl.when(sid < _NS - 1)
    def _():
        pltpu.sync_copy(zeros_hbm.at[pl.ds(zoff, slab)],
                        acc_sh.at[pl.ds(zoff, slab)])

    @pl.when(sid == _NS - 1)
    def _():
        pltpu.sync_copy(zeros_hbm.at[pl.ds((_NS - 1) * slab, last)],
                        acc_sh.at[pl.ds((_NS - 1) * slab, last)])

    plsc.subcore_barrier()

    wid = sid * _NC + cid
    cstart = wid * _SPW

    def _m(k):
        return m_hbm.at[pl.ds((cstart + k) * _GC, _GC)]

    pltpu.sync_copy(dst2d_hbm.at[pl.ds(cstart, _SPW)], idx_v)

    def _run(nch):
        pltpu.async_copy(_m(0), rows0, sem0)

        @pl.loop(0, nch, step=2)
        def _(k):
            pltpu.make_async_copy(_m(k), rows0, sem0).wait()
            pltpu.async_copy(_m(k + 1), rows1, sem1)
            pltpu.sync_copy(rows0, acc_sh.at[idx_v.at[k]], add=True)

            @pl.when(k + 2 < nch)
            def _():
                pltpu.async_copy(_m(k + 2), rows0, sem0)

            pltpu.make_async_copy(_m(k + 1), rows1, sem1).wait()
            pltpu.sync_copy(rows1, acc_sh.at[idx_v.at[k + 1]], add=True)

    @pl.when(wid < _NW - 1)
    def _():
        _run(_SPW)

    @pl.when(wid == _NW - 1)
    def _():
        _run(_SLAST)

    plsc.subcore_barrier()
    doff = sid * slab

    @pl.when(sid < _NS - 1)
    def _():
        pltpu.sync_copy(acc_sh.at[pl.ds(doff, slab)],
                        out_hbm.at[pl.ds(cid * _N + doff, slab)])

    @pl.when(sid == _NS - 1)
    def _():
        pltpu.sync_copy(acc_sh.at[pl.ds((_NS - 1) * slab, last)],
                        out_hbm.at[pl.ds(cid * _N + (_NS - 1) * slab, last)])


def _sc_segment_sum(m, dst2d, zeros):
    return pl.kernel(
        _sc_scatter_body,
        out_type=jax.ShapeDtypeStruct((2 * _N, _LD), jnp.float32),
        mesh=_sc_mesh(),
        scratch_types=[
            pltpu.VMEM((_SPW, _GC), jnp.int32),
            pltpu.VMEM((_GC, _LD), jnp.float32),
            pltpu.VMEM((_GC, _LD), jnp.float32),
            pltpu.VMEM_SHARED((_N, _LD), jnp.float32),
            pltpu.SemaphoreType.DMA,
            pltpu.SemaphoreType.DMA,
        ],
    )(m, dst2d, zeros)


def kernel(x, edge_index, edge_attr, acceleration, Wn, bn, Wa, ba, We, be,
           Wm1, bm1, Wm2, bm2, Wu1, bu1, Wu2, bu2, ln_g, ln_b, Wd, bd):
    src = edge_index[0]
    dst = edge_index[1]
    idx2 = jnp.concatenate([src, dst + _N])
    dst2d = jnp.pad(dst.reshape(_SCH, _GC),
                    ((0, _NW * _SPW - _SCH), (0, 0)))
    zeros = jnp.zeros((_N, _LD), jnp.float32)
    h = _encode(x, Wn, bn)
    eh = _edge_encode(edge_attr, We, be)
    acc = acceleration
    rem = sum(_BITWAVES)
    logits_list = []
    for i, bw in enumerate(_BITWAVES):
        A = Wm1[i, 0 * _LD:1 * _LD]
        B = Wm1[i, 1 * _LD:2 * _LD]
        C = Wm1[i, 2 * _LD:3 * _LD]
        D = Wm1[i, 3 * _LD:4 * _LD]
        F = Wm1[i, 4 * _LD:5 * _LD]
        ST, acc_h = _pre(h, acc, Wa, ba, A, B, C, D)
        Gst = _sc_gather(ST, idx2)
        m = _edge_mlp(Gst, eh, F, bm1[i], Wm2[i], bm2[i])
        agg_p = _sc_segment_sum(m, dst2d, zeros)
        scale = float(2.0 ** (-(rem + _BQ)))
        h, acc, logits = _post(
            h, acc_h, acc, agg_p,
            Wu1[i, 0 * _LD:1 * _LD], Wu1[i, 1 * _LD:2 * _LD],
            Wu1[i, 2 * _LD:3 * _LD], bu1[i], Wu2[i], bu2[i],
            ln_g[i], ln_b[i], Wd[i], bd[i], scale)
        logits_list.append(logits.reshape(_N, _OUT, _R))
        rem -= bw
    return (acc, jnp.stack(logits_list))


# speedup vs baseline: 1.0073x; 1.0073x over previous
import functools

import jax
import jax.numpy as jnp
from jax import lax
from jax.experimental import pallas as pl
from jax.experimental.pallas import tpu as pltpu
from jax.experimental.pallas import tpu_sc as plsc

_N = 10000
_E = 320000
_LD = 128
_OUT = 3
_R = 16
_BITWAVES = (4, 4)
_BQ = 4

_INTERPRET = False


def _dot(a, b):
    return jnp.dot(a, b, preferred_element_type=jnp.float32)


def _encode_body(x_ref, Wn_ref, bn_ref, h_ref):
    h_ref[...] = _dot(x_ref[...], Wn_ref[...]) + bn_ref[...]


def _encode(x, Wn, bn):
    return pl.pallas_call(
        _encode_body,
        out_shape=jax.ShapeDtypeStruct((_N, _LD), jnp.float32),
        interpret=_INTERPRET,
    )(x, Wn, bn.reshape(1, _LD))


def _pre_body(h_ref, acc_ref, Wa_ref, ba_ref, A_ref, B_ref, C_ref, D_ref,
              ST_ref, acch_ref):
    h = h_ref[...]
    acc_h = _dot(acc_ref[...], Wa_ref[...]) + ba_ref[...]
    acch_ref[...] = acc_h
    ST_ref[0:_N, :] = _dot(h, A_ref[...]) + _dot(acc_h, C_ref[...])
    ST_ref[_N:2 * _N, :] = _dot(h, B_ref[...]) + _dot(acc_h, D_ref[...])


def _pre(h, acc, Wa, ba, A, B, C, D):
    return pl.pallas_call(
        _pre_body,
        out_shape=(jax.ShapeDtypeStruct((2 * _N, _LD), jnp.float32),
                   jax.ShapeDtypeStruct((_N, _LD), jnp.float32)),
        interpret=_INTERPRET,
    )(h, acc, Wa, ba.reshape(1, _LD), A, B, C, D)


def _edge_encode_body(ea_ref, We_ref, be_ref, eh_ref):
    eh_ref[...] = _dot(ea_ref[...], We_ref[...]) + be_ref[...]


def _edge_encode(edge_attr, We, be):
    BE = 2560
    return pl.pallas_call(
        _edge_encode_body,
        grid=(_E // BE,),
        in_specs=[
            pl.BlockSpec((BE, 16), lambda i: (i, 0)),
            pl.BlockSpec((16, _LD), lambda i: (0, 0)),
            pl.BlockSpec((1, _LD), lambda i: (0, 0)),
        ],
        out_specs=pl.BlockSpec((BE, _LD), lambda i: (i, 0)),
        out_shape=jax.ShapeDtypeStruct((_E, _LD), jnp.float32),
        interpret=_INTERPRET,
    )(edge_attr, We, be.reshape(1, _LD))


def _edge_body(GS_ref, GT_ref, eh_ref, F_ref, bm1_ref, Wm2_ref, bm2_ref,
               m_ref):
    hidden = jnp.maximum(
        GS_ref[...] + GT_ref[...] + _dot(eh_ref[...], F_ref[...])
        + bm1_ref[...], 0.0)
    m_ref[...] = jnp.maximum(_dot(hidden, Wm2_ref[...]) + bm2_ref[...], 0.0)


def _edge_mlp(Gst, eh, F, bm1, Wm2, bm2):
    BE = 2560
    nblk = _E // BE
    grid = (nblk,)
    return pl.pallas_call(
        _edge_body,
        grid=grid,
        in_specs=[
            pl.BlockSpec((BE, _LD), lambda i: (i, 0)),
            pl.BlockSpec((BE, _LD), lambda i: (i + nblk, 0)),
            pl.BlockSpec((BE, _LD), lambda i: (i, 0)),
            pl.BlockSpec((_LD, _LD), lambda i: (0, 0)),
            pl.BlockSpec((1, _LD), lambda i: (0, 0)),
            pl.BlockSpec((_LD, _LD), lambda i: (0, 0)),
            pl.BlockSpec((1, _LD), lambda i: (0, 0)),
        ],
        out_specs=pl.BlockSpec((BE, _LD), lambda i: (i, 0)),
        out_shape=jax.ShapeDtypeStruct((_E, _LD), jnp.float32),
        interpret=_INTERPRET,
    )(Gst, Gst, eh, F, bm1.reshape(1, _LD), Wm2, bm2.reshape(1, _LD))


def _post_body(h_ref, acch_ref, acc_ref, aggp_ref, Wu1a_ref, Wu1b_ref,
               Wu1c_ref, bu1_ref, Wu2_ref, bu2_ref, g_ref, b_ref, Wd_ref,
               bd_ref, h1_ref, acc1_ref, logits_ref, *, scale):
    agg = aggp_ref[0:_N, :] + aggp_ref[_N:2 * _N, :]
    u = jnp.maximum(
        _dot(h_ref[...], Wu1a_ref[...]) + _dot(acch_ref[...], Wu1b_ref[...])
        + _dot(agg, Wu1c_ref[...]) + bu1_ref[...], 0.0)
    hn = _dot(u, Wu2_ref[...]) + bu2_ref[...]
    mu = jnp.mean(hn, axis=1, keepdims=True)
    hc = hn - mu
    var = jnp.mean(hc * hc, axis=1, keepdims=True)
    hn = hc / jnp.sqrt(var + 1e-5) * g_ref[...] + b_ref[...]
    h1_ref[...] = hn
    logits = _dot(hn, Wd_ref[...]) + bd_ref[...]
    logits_ref[...] = logits
    cols = []
    for j in range(_OUT):
        lj = logits[:, j * _R:(j + 1) * _R]
        mx = jnp.max(lj, axis=1, keepdims=True)
        iota = lax.broadcasted_iota(jnp.int32, lj.shape, 1)
        idx = jnp.min(jnp.where(lj == mx, iota, _R), axis=1, keepdims=True)
        cols.append(idx)
    bits = jnp.concatenate(cols, axis=1).astype(jnp.float32)
    acc1_ref[...] = acc_ref[...] + bits * scale


def _post(h, acc_h, acc, agg_p, Wu1a, Wu1b, Wu1c, bu1, Wu2, bu2, g, b, Wd, bd,
          scale):
    outs = (
        jax.ShapeDtypeStruct((_N, _LD), jnp.float32),
        jax.ShapeDtypeStruct((_N, _OUT), jnp.float32),
        jax.ShapeDtypeStruct((_N, _OUT * _R), jnp.float32),
    )
    return pl.pallas_call(
        functools.partial(_post_body, scale=scale),
        out_shape=outs,
        interpret=_INTERPRET,
    )(h, acc_h, acc, agg_p, Wu1a, Wu1b, Wu1c, bu1.reshape(1, _LD), Wu2,
      bu2.reshape(1, _LD), g.reshape(1, _LD), b.reshape(1, _LD), Wd,
      bd.reshape(1, _OUT * _R))


_NC = 2
_NS = 16
_NW = _NC * _NS
_GC = 128


def _sc_mesh():
    return plsc.VectorSubcoreMesh(core_axis_name="c", subcore_axis_name="s")


_GCH = (2 * _E) // _GC
_GFULL = _GCH // _NW
_GEXTRA = _GCH - _GFULL * _NW


_GNB = 4


def _sc_gather_body(table_hbm, idx_hbm, out_hbm, idx_v, rows0, rows1, rows2,
                    rows3, sem0, sem1, sem2, sem3):
    rows = (rows0, rows1, rows2, rows3)
    sems = (sem0, sem1, sem2, sem3)
    wid = lax.axis_index("s") * _NC + lax.axis_index("c")
    cstart = wid * _GFULL + jnp.minimum(wid, _GEXTRA)
    pltpu.sync_copy(idx_hbm.at[pl.ds(cstart * _GC, _GFULL * _GC)],
                    idx_v.at[pl.ds(0, _GFULL * _GC)])

    @pl.when(wid < _GEXTRA)
    def _():
        pltpu.sync_copy(idx_hbm.at[pl.ds((cstart + _GFULL) * _GC, _GC)],
                        idx_v.at[pl.ds(_GFULL * _GC, _GC)])

    def _idx(k):
        return idx_v.at[pl.ds(k * _GC, _GC)]

    def _out(k):
        return out_hbm.at[pl.ds((cstart + k) * _GC, _GC)]

    for b in range(_GNB - 1):
        pltpu.async_copy(table_hbm.at[_idx(b)], rows[b], sems[b])

    @pl.loop(0, _GFULL, step=_GNB)
    def _(k):
        for b in range(_GNB):
            jb = k + b
            pltpu.make_async_copy(table_hbm.at[_idx(jb)], rows[b],
                                  sems[b]).wait()
            nb = (b + _GNB - 1) % _GNB

            @pl.when(jb + _GNB - 1 < _GFULL)
            def _(jb=jb, nb=nb):
                pltpu.async_copy(table_hbm.at[_idx(jb + _GNB - 1)], rows[nb],
                                 sems[nb])

            pltpu.sync_copy(rows[b], _out(jb))

    @pl.when(wid < _GEXTRA)
    def _():
        pltpu.async_copy(table_hbm.at[_idx(_GFULL)], rows0, sem0).wait()
        pltpu.sync_copy(rows0, _out(_GFULL))


def _sc_gather(ST, idx2):
    return pl.kernel(
        _sc_gather_body,
        out_type=jax.ShapeDtypeStruct((2 * _E, _LD), jnp.float32),
        mesh=_sc_mesh(),
        scratch_types=[
            pltpu.VMEM(((_GFULL + 1) * _GC,), jnp.int32),
            pltpu.VMEM((_GC, _LD), jnp.float32),
            pltpu.VMEM((_GC, _LD), jnp.float32),
            pltpu.VMEM((_GC, _LD), jnp.float32),
            pltpu.VMEM((_GC, _LD), jnp.float32),
            pltpu.SemaphoreType.DMA,
            pltpu.SemaphoreType.DMA,
            pltpu.SemaphoreType.DMA,
            pltpu.SemaphoreType.DMA,
        ],
    )(ST, idx2)


_SCH = _E // _GC
_SPW = 80
_SLAST = _SCH - _SPW * (_NW - 1)


def _sc_scatter_body(m_hbm, dst2d_hbm, zeros_hbm, out_hbm, idx_v, rows0,
                     rows1, acc_sh, sem0, sem1):
    cid = lax.axis_index("c")
    sid = lax.axis_index("s")
    slab = 632
    last = _N - (_NS - 1) * slab
    zoff = sid * slab

    @pl.when(sid < _NS - 1)
    def _():
        pltpu.sync_copy(zeros_hbm.at[pl.ds(zoff, slab)],
                        acc_sh.at[pl.ds(zoff, slab)])

    @pl.when(sid == _NS - 1)
    def _():
        pltpu.sync_copy(zeros_hbm.at[pl.ds((_NS - 1) * slab, last)],
                        acc_sh.at[pl.ds((_NS - 1) * slab, last)])

    plsc.subcore_barrier()

    wid = sid * _NC + cid
    cstart = wid * _SPW

    def _m(k):
        return m_hbm.at[pl.ds((cstart + k) * _GC, _GC)]

    pltpu.sync_copy(dst2d_hbm.at[pl.ds(cstart, _SPW)], idx_v)

    def _run(nch):
        pltpu.async_copy(_m(0), rows0, sem0)

        @pl.loop(0, nch, step=2)
        def _(k):
            pltpu.make_async_copy(_m(k), rows0, sem0).wait()
            pltpu.async_copy(_m(k + 1), rows1, sem1)
            pltpu.sync_copy(rows0, acc_sh.at[idx_v.at[k]], add=True)

            @pl.when(k + 2 < nch)
            def _():
                pltpu.async_copy(_m(k + 2), rows0, sem0)

            pltpu.make_async_copy(_m(k + 1), rows1, sem1).wait()
            pltpu.sync_copy(rows1, acc_sh.at[idx_v.at[k + 1]], add=True)

    @pl.when(wid < _NW - 1)
    def _():
        _run(_SPW)

    @pl.when(wid == _NW - 1)
    def _():
        _run(_SLAST)

    plsc.subcore_barrier()
    doff = sid * slab

    @pl.when(sid < _NS - 1)
    def _():
        pltpu.sync_copy(acc_sh.at[pl.ds(doff, slab)],
                        out_hbm.at[pl.ds(cid * _N + doff, slab)])

    @pl.when(sid == _NS - 1)
    def _():
        pltpu.sync_copy(acc_sh.at[pl.ds((_NS - 1) * slab, last)],
                        out_hbm.at[pl.ds(cid * _N + (_NS - 1) * slab, last)])


def _sc_segment_sum(m, dst2d, zeros):
    return pl.kernel(
        _sc_scatter_body,
        out_type=jax.ShapeDtypeStruct((2 * _N, _LD), jnp.float32),
        mesh=_sc_mesh(),
        scratch_types=[
            pltpu.VMEM((_SPW, _GC), jnp.int32),
            pltpu.VMEM((_GC, _LD), jnp.float32),
            pltpu.VMEM((_GC, _LD), jnp.float32),
            pltpu.VMEM_SHARED((_N, _LD), jnp.float32),
            pltpu.SemaphoreType.DMA,
            pltpu.SemaphoreType.DMA,
        ],
    )(m, dst2d, zeros)


def kernel(x, edge_index, edge_attr, acceleration, Wn, bn, Wa, ba, We, be,
           Wm1, bm1, Wm2, bm2, Wu1, bu1, Wu2, bu2, ln_g, ln_b, Wd, bd):
    src = edge_index[0]
    dst = edge_index[1]
    idx2 = jnp.concatenate([src, dst + _N])
    dst2d = jnp.pad(dst.reshape(_SCH, _GC),
                    ((0, _NW * _SPW - _SCH), (0, 0)))
    zeros = jnp.zeros((_N, _LD), jnp.float32)
    h = _encode(x, Wn, bn)
    eh = _edge_encode(edge_attr, We, be)
    acc = acceleration
    rem = sum(_BITWAVES)
    logits_list = []
    for i, bw in enumerate(_BITWAVES):
        A = Wm1[i, 0 * _LD:1 * _LD]
        B = Wm1[i, 1 * _LD:2 * _LD]
        C = Wm1[i, 2 * _LD:3 * _LD]
        D = Wm1[i, 3 * _LD:4 * _LD]
        F = Wm1[i, 4 * _LD:5 * _LD]
        ST, acc_h = _pre(h, acc, Wa, ba, A, B, C, D)
        Gst = _sc_gather(ST, idx2)
        m = _edge_mlp(Gst, eh, F, bm1[i], Wm2[i], bm2[i])
        agg_p = _sc_segment_sum(m, dst2d, zeros)
        scale = float(2.0 ** (-(rem + _BQ)))
        h, acc, logits = _post(
            h, acc_h, acc, agg_p,
            Wu1[i, 0 * _LD:1 * _LD], Wu1[i, 1 * _LD:2 * _LD],
            Wu1[i, 2 * _LD:3 * _LD], bu1[i], Wu2[i], bu2[i],
            ln_g[i], ln_b[i], Wd[i], bd[i], scale)
        logits_list.append(logits.reshape(_N, _OUT, _R))
        rem -= bw
    return (acc, jnp.stack(logits_list))
